# baseline (speedup 1.0000x reference)
import functools

import jax
import jax.numpy as jnp
from jax import lax
from jax.experimental import pallas as pl
from jax.experimental.pallas import tpu as pltpu
from jax.experimental.pallas import tpu_sc as plsc

MAX_DEG = 512
D = 128
N_ROWS = 16 * 4096
N_WORKERS = 32
ROWS_PER_WORKER = N_ROWS // N_WORKERS
CHUNK = 128
N_CHUNKS = ROWS_PER_WORKER // CHUNK
IDX_ROWS = ROWS_PER_WORKER // D
L = 16


def _body(x_hbm, iidx_hbm, oidx_hbm, itab_hbm, otab_hbm, out_hbm,
          iidx_v, oidx_v, itab_sh, otab_sh,
          acc0, acc1, acc2, acc3, acc4, acc5,
          sx0, sx1, sx2, sx3, sx4, sx5, sa0, sa1, sa2, sa3, sa4, sa5,
          sb0, sb1, sb2, sb3, sb4, sb5, so0, so1, so2, so3, so4, so5):
    sid = lax.axis_index("s")
    wid = sid * 2 + lax.axis_index("c")
    @pl.when(sid == 0)
    def _():
        pltpu.sync_copy(itab_hbm, itab_sh)

    @pl.when(sid == 1)
    def _():
        pltpu.sync_copy(otab_hbm, otab_sh)

    plsc.subcore_barrier()
    idx_row0 = wid * IDX_ROWS
    pltpu.sync_copy(iidx_hbm.at[pl.ds(idx_row0, IDX_ROWS)], iidx_v)
    pltpu.sync_copy(oidx_hbm.at[pl.ds(idx_row0, IDX_ROWS)], oidx_v)
    for i in range(IDX_ROWS):
        for j in range(D // L):
            sl = (i, pl.ds(j * L, L))
            iidx_v[sl] = jnp.clip(iidx_v[sl], 0, MAX_DEG)
            oidx_v[sl] = jnp.clip(oidx_v[sl], 0, MAX_DEG)

    acc = (acc0, acc1, acc2, acc3, acc4, acc5)
    sx = (sx0, sx1, sx2, sx3, sx4, sx5)
    sa = (sa0, sa1, sa2, sa3, sa4, sa5)
    sb = (sb0, sb1, sb2, sb3, sb4, sb5)
    so = (so0, so1, so2, so3, so4, so5)
    NBUF = 6
    row0 = wid * ROWS_PER_WORKER

    def x_src(c):
        return x_hbm.at[pl.ds(row0 + c * CHUNK, CHUNK)]

    def out_dst(c):
        return out_hbm.at[pl.ds(row0 + c * CHUNK, CHUNK)]

    for c in range(N_CHUNKS + 3):
        if c < N_CHUNKS:
            b = c % NBUF
            if c >= NBUF:
                pltpu.make_async_copy(acc[b], out_dst(c - NBUF), so[b]).wait()
            pltpu.async_copy(x_src(c), acc[b], sx[b])
        g = c - 2
        if 0 <= g < N_CHUNKS:
            b = g % NBUF
            pltpu.make_async_copy(x_src(g), acc[b], sx[b]).wait()
            pltpu.async_copy(itab_sh.at[iidx_v.at[g]], acc[b], sa[b], add=True)
            pltpu.async_copy(otab_sh.at[oidx_v.at[g]], acc[b], sb[b], add=True)
        s = c - 3
        if 0 <= s < N_CHUNKS:
            b = s % NBUF
            pltpu.make_async_copy(itab_sh.at[iidx_v.at[s]], acc[b], sa[b]).wait()
            pltpu.make_async_copy(otab_sh.at[oidx_v.at[s]], acc[b], sb[b]).wait()
            pltpu.async_copy(acc[b], out_dst(s), so[b])
    for s in range(N_CHUNKS - NBUF, N_CHUNKS):
        b = s % NBUF
        pltpu.make_async_copy(acc[b], out_dst(s), so[b]).wait()


@functools.partial(
    pl.kernel,
    mesh=plsc.VectorSubcoreMesh(core_axis_name="c", subcore_axis_name="s"),
    out_type=jax.ShapeDtypeStruct((N_ROWS, D), jnp.float32),
    scratch_types=[
        pltpu.VMEM((IDX_ROWS, D), jnp.int32),
        pltpu.VMEM((IDX_ROWS, D), jnp.int32),
        pltpu.VMEM_SHARED((MAX_DEG + 1, D), jnp.float32),
        pltpu.VMEM_SHARED((MAX_DEG + 1, D), jnp.float32),
        pltpu.VMEM((CHUNK, D), jnp.float32),
        pltpu.VMEM((CHUNK, D), jnp.float32),
        pltpu.VMEM((CHUNK, D), jnp.float32),
        pltpu.VMEM((CHUNK, D), jnp.float32),
        pltpu.VMEM((CHUNK, D), jnp.float32),
        pltpu.VMEM((CHUNK, D), jnp.float32),
    ] + [pltpu.SemaphoreType.DMA] * 24,
)
def _sc_call(*refs):
    _body(*refs)


def kernel(x, in_degrees, out_degrees, in_table, out_table):
    xf = x.reshape(N_ROWS, D)
    ii = in_degrees.astype(jnp.int32).reshape(N_ROWS // D, D)
    oo = out_degrees.astype(jnp.int32).reshape(N_ROWS // D, D)
    out = _sc_call(xf, ii, oo, in_table, out_table)
    return out.reshape(x.shape)


# speedup vs baseline: 1.0379x; 1.0379x over previous
import functools

import jax
import jax.numpy as jnp
from jax import lax
from jax.experimental import pallas as pl
from jax.experimental.pallas import tpu as pltpu
from jax.experimental.pallas import tpu_sc as plsc

MAX_DEG = 512
D = 128
N_ROWS = 16 * 4096
N_WORKERS = 32
ROWS_PER_WORKER = N_ROWS // N_WORKERS
CHUNK = 128
N_CHUNKS = ROWS_PER_WORKER // CHUNK
IDX_ROWS = ROWS_PER_WORKER // D
L = 16


def _body(x_hbm, iidx_hbm, oidx_hbm, itab_hbm, otab_hbm, out_hbm,
          iidx_v, oidx_v, itab_sh, otab_sh, acc0, acc1, acc2, acc3,
          sx0, sx1, sx2, sx3, sa0, sa1, sa2, sa3,
          sb0, sb1, sb2, sb3, so0, so1, so2, so3):
    sid = lax.axis_index("s")
    wid = sid * 2 + lax.axis_index("c")
    row0 = wid * ROWS_PER_WORKER

    acc = (acc0, acc1, acc2, acc3)
    sx = (sx0, sx1, sx2, sx3)
    sa = (sa0, sa1, sa2, sa3)
    sb = (sb0, sb1, sb2, sb3)
    so = (so0, so1, so2, so3)
    NBUF = 4

    def x_src(c):
        return x_hbm.at[pl.ds(row0 + c * CHUNK, CHUNK)]

    def out_dst(c):
        return out_hbm.at[pl.ds(row0 + c * CHUNK, CHUNK)]

    for c in range(NBUF):
        pltpu.async_copy(x_src(c), acc[c], sx[c])

    @pl.when(sid == 0)
    def _():
        pltpu.sync_copy(itab_hbm, itab_sh)

    @pl.when(sid == 1)
    def _():
        pltpu.sync_copy(otab_hbm, otab_sh)

    idx_row0 = wid * IDX_ROWS
    pltpu.sync_copy(iidx_hbm.at[pl.ds(idx_row0, IDX_ROWS)], iidx_v)
    pltpu.sync_copy(oidx_hbm.at[pl.ds(idx_row0, IDX_ROWS)], oidx_v)
    for i in range(IDX_ROWS):
        for j in range(D // L):
            sl = (i, pl.ds(j * L, L))
            iidx_v[sl] = jnp.clip(iidx_v[sl], 0, MAX_DEG)
            oidx_v[sl] = jnp.clip(oidx_v[sl], 0, MAX_DEG)
    plsc.subcore_barrier()

    for c in range(N_CHUNKS + 2):
        if NBUF <= c < N_CHUNKS:
            b = c % NBUF
            pltpu.make_async_copy(acc[b], out_dst(c - NBUF), so[b]).wait()
            pltpu.async_copy(x_src(c), acc[b], sx[b])
        g = c - 1
        if 0 <= g < N_CHUNKS:
            b = g % NBUF
            pltpu.make_async_copy(x_src(g), acc[b], sx[b]).wait()
            pltpu.async_copy(itab_sh.at[iidx_v.at[g]], acc[b], sa[b], add=True)
            pltpu.async_copy(otab_sh.at[oidx_v.at[g]], acc[b], sb[b], add=True)
        s = c - 2
        if 0 <= s < N_CHUNKS:
            b = s % NBUF
            pltpu.make_async_copy(itab_sh.at[iidx_v.at[s]], acc[b], sa[b]).wait()
            pltpu.make_async_copy(otab_sh.at[oidx_v.at[s]], acc[b], sb[b]).wait()
            pltpu.async_copy(acc[b], out_dst(s), so[b])
    for s in range(N_CHUNKS - NBUF, N_CHUNKS):
        b = s % NBUF
        pltpu.make_async_copy(acc[b], out_dst(s), so[b]).wait()


@functools.partial(
    pl.kernel,
    mesh=plsc.VectorSubcoreMesh(core_axis_name="c", subcore_axis_name="s"),
    out_type=jax.ShapeDtypeStruct((N_ROWS, D), jnp.float32),
    scratch_types=[
        pltpu.VMEM((IDX_ROWS, D), jnp.int32),
        pltpu.VMEM((IDX_ROWS, D), jnp.int32),
        pltpu.VMEM_SHARED((MAX_DEG + 1, D), jnp.float32),
        pltpu.VMEM_SHARED((MAX_DEG + 1, D), jnp.float32),
        pltpu.VMEM((CHUNK, D), jnp.float32),
        pltpu.VMEM((CHUNK, D), jnp.float32),
        pltpu.VMEM((CHUNK, D), jnp.float32),
        pltpu.VMEM((CHUNK, D), jnp.float32),
    ] + [pltpu.SemaphoreType.DMA] * 16,
)
def _sc_call(*refs):
    _body(*refs)


def kernel(x, in_degrees, out_degrees, in_table, out_table):
    xf = x.reshape(N_ROWS, D)
    ii = in_degrees.astype(jnp.int32).reshape(N_ROWS // D, D)
    oo = out_degrees.astype(jnp.int32).reshape(N_ROWS // D, D)
    out = _sc_call(xf, ii, oo, in_table, out_table)
    return out.reshape(x.shape)
